# baseline (speedup 1.0000x reference)
import jax
import jax.numpy as jnp
from jax import lax
from jax.experimental import pallas as pl
from jax.experimental.pallas import tpu as pltpu
from jax.experimental.pallas import tpu_sc as plsc

B, M, HW, PATCH, D, C = 4, 8, 512, 16, 768, 8
G = HW // PATCH
P = G * G
BM = B * M
F32 = jnp.float32

_NC, _NS = 2, 16
_ROWS = P // M


def _fused_body(mask_ref, emb_ref, cat_ref,
                mref_ref, flat_ref, cnt_ref, scat_ref, pooled_s):
    i = pl.program_id(0)
    x = mask_ref[0]
    inv = 1.0 - x
    a = (lax.broadcasted_iota(jnp.int32, (HW, G), 0) // PATCH ==
         lax.broadcasted_iota(jnp.int32, (HW, G), 1)).astype(F32)
    t = jnp.dot(inv, a, preferred_element_type=F32)
    s = lax.dot_general(a, t, (((0,), (0,)), ((), ())),
                        preferred_element_type=F32)
    sb = jnp.where(s == 0.0, 1.0, 0.0)
    e_sel = (lax.broadcasted_iota(jnp.int32, (G, P), 1) % G ==
             lax.broadcasted_iota(jnp.int32, (G, P), 0)).astype(F32)
    k_rep = jnp.dot(sb, e_sel, preferred_element_type=F32)
    r_msk = (lax.broadcasted_iota(jnp.int32, (G, P), 1) // G ==
             lax.broadcasted_iota(jnp.int32, (G, P), 0)).astype(F32)
    pooled_s[pl.ds(i, 1)] = jnp.sum(k_rep * r_msk, axis=0, keepdims=True)

    @pl.when(i == BM - 1)
    def _stats():
        pooled = pooled_s[...]
        keep = (pooled != 0.0).astype(F32)
        cat_row = cat_ref[...]

        keep4 = jnp.concatenate([keep] * B, axis=1)
        bmask = (lax.broadcasted_iota(jnp.int32, (BM, B * P), 0) // M ==
                 lax.broadcasted_iota(jnp.int32, (BM, B * P), 1) // P
                 ).astype(F32)
        sum_per_mask = jnp.dot(keep4 * bmask, emb_ref[...],
                               preferred_element_type=F32,
                               precision=lax.Precision.HIGHEST)

        cnt_col = jnp.dot(keep, jnp.ones((P, 1), F32),
                          preferred_element_type=F32)

        cid = lax.broadcasted_iota(jnp.int32, (C, BM), 0)
        onehot = (jnp.broadcast_to(cat_row, (C, BM)) == cid).astype(F32)

        cat_sum = jnp.dot(onehot, sum_per_mask, preferred_element_type=F32,
                          precision=lax.Precision.HIGHEST)
        cat_cnt = jnp.dot(onehot, cnt_col, preferred_element_type=F32)
        mean = cat_sum / jnp.maximum(cat_cnt, 1.0)
        nrm = jnp.sqrt(jnp.sum(mean * mean, axis=1, keepdims=True))
        mref_ref[...] = mean / jnp.maximum(nrm, 1e-12)

        counts_col = jnp.dot(onehot, jnp.ones((BM, 1), F32),
                             preferred_element_type=F32)
        cnt_ref[...] = counts_col.astype(jnp.int32)

        lessc = jnp.dot((lax.broadcasted_iota(jnp.int32, (C, C), 1) <
                         lax.broadcasted_iota(jnp.int32, (C, C), 0)
                         ).astype(F32),
                        counts_col, preferred_element_type=F32)
        less_row = jnp.sum(onehot * lessc, axis=0, keepdims=True)
        u_strict = (lax.broadcasted_iota(jnp.int32, (BM, BM), 0) <
                    lax.broadcasted_iota(jnp.int32, (BM, BM), 1)).astype(F32)
        prefix = jnp.dot(onehot, u_strict, preferred_element_type=F32)
        eqb_row = jnp.sum(onehot * prefix, axis=0, keepdims=True)
        rank_row = less_row + eqb_row

        rowj = lax.broadcasted_iota(jnp.int32, (BM, BM), 0)
        pm = (jnp.broadcast_to(rank_row, (BM, BM)) ==
              rowj.astype(F32)).astype(F32)
        flat_ref[...] = jnp.dot(pm, pooled, preferred_element_type=F32)

        rank_b = jnp.dot(
            (lax.broadcasted_iota(jnp.int32, (B, BM), 1) //
             M == lax.broadcasted_iota(jnp.int32, (B, BM), 0)).astype(F32) *
            jnp.broadcast_to(rank_row, (B, BM)),
            (lax.broadcasted_iota(jnp.int32, (BM, M), 0) % M ==
             lax.broadcasted_iota(jnp.int32, (BM, M), 1)).astype(F32),
            preferred_element_type=F32)
        rep_row = (lax.broadcasted_iota(jnp.int32, (BM, B), 0) // M ==
                   lax.broadcasted_iota(jnp.int32, (BM, B), 1)).astype(F32)
        rep_col = (lax.broadcasted_iota(jnp.int32, (M, P), 1) // _ROWS ==
                   lax.broadcasted_iota(jnp.int32, (M, P), 0)).astype(F32)
        rank_val = jnp.dot(
            jnp.dot(rep_row, rank_b, preferred_element_type=F32),
            rep_col, preferred_element_type=F32)
        w_iota = lax.broadcasted_iota(jnp.int32, (BM, P), 0)
        q_iota = lax.broadcasted_iota(jnp.int32, (BM, P), 1)
        scat_ref[...] = (rank_val.astype(jnp.int32) * P +
                         (w_iota % M) * _ROWS + q_iota % _ROWS)


def _fused_call(masks_flat, emb2d, cat_2d):
    return pl.pallas_call(
        _fused_body,
        grid=(BM,),
        in_specs=[
            pl.BlockSpec((1, HW, HW), lambda i: (i, 0, 0)),
            pl.BlockSpec((B * P, D), lambda i: (0, 0)),
            pl.BlockSpec((1, BM), lambda i: (0, 0)),
        ],
        out_specs=(
            pl.BlockSpec((C, D), lambda i: (0, 0)),
            pl.BlockSpec((BM, P), lambda i: (0, 0)),
            pl.BlockSpec((C, 1), lambda i: (0, 0)),
            pl.BlockSpec((BM, P), lambda i: (0, 0)),
        ),
        out_shape=(
            jax.ShapeDtypeStruct((C, D), F32),
            jax.ShapeDtypeStruct((BM, P), F32),
            jax.ShapeDtypeStruct((C, 1), jnp.int32),
            jax.ShapeDtypeStruct((BM, P), jnp.int32),
        ),
        scratch_shapes=[pltpu.VMEM((BM, P), F32)],
    )(masks_flat, emb2d, cat_2d)


def _gather_body(idx_hbm, emb_hbm, out_hbm, idx_v, buf, isem, osem):
    wid = lax.axis_index("s") * _NC + lax.axis_index("c")
    pltpu.sync_copy(idx_hbm.at[wid], idx_v)
    base = (wid // M) * P + (wid % M) * _ROWS
    pltpu.async_copy(emb_hbm.at[pl.ds(base, _ROWS)], buf, isem).wait()
    hs = []
    for j in range(M):
        hs.append(pltpu.async_copy(buf, out_hbm.at[idx_v.at[j]], osem))
    for h in hs:
        h.wait()


def _gather_call(idx3d, emb2d):
    mesh = plsc.VectorSubcoreMesh(core_axis_name="c", subcore_axis_name="s")
    fn = pl.kernel(
        _gather_body,
        mesh=mesh,
        out_type=jax.ShapeDtypeStruct((BM * P, D), F32),
        scratch_types=[
            pltpu.VMEM((M, _ROWS), jnp.int32),
            pltpu.VMEM((_ROWS, D), F32),
            pltpu.SemaphoreType.DMA,
            pltpu.SemaphoreType.DMA,
        ],
    )
    return fn(idx3d, emb2d)


def kernel(embeddings, masks, category_ids):
    cat_flat = category_ids.reshape(BM)
    masks_flat = masks.reshape(BM, HW, HW)
    emb2d = embeddings.reshape(B * P, D)

    masked_ref, flat_sorted, counts_col, scat = _fused_call(
        masks_flat, emb2d, cat_flat.reshape(1, BM))
    ref_emb = _gather_call(scat.reshape(BM, M, _ROWS), emb2d)

    return (masked_ref,
            flat_sorted.reshape(BM * P),
            ref_emb,
            counts_col.reshape(C))


# speedup vs baseline: 1.0027x; 1.0027x over previous
import functools

import jax
import jax.numpy as jnp
from jax import lax
from jax.experimental import pallas as pl
from jax.experimental.pallas import tpu as pltpu
from jax.experimental.pallas import tpu_sc as plsc

B, M, HW, PATCH, D, C = 4, 8, 512, 16, 768, 8
G = HW // PATCH
P = G * G
BM = B * M
F32 = jnp.float32


def _pool_body(mask_ref, pooled_ref):
    x = mask_ref[0]
    inv = 1.0 - x
    r = lax.broadcasted_iota(jnp.int32, (HW, G), 0) // PATCH
    c = lax.broadcasted_iota(jnp.int32, (HW, G), 1)
    a = (r == c).astype(F32)
    t = jnp.dot(inv, a, preferred_element_type=F32)
    s = lax.dot_general(a, t, (((0,), (0,)), ((), ())),
                        preferred_element_type=F32)
    pooled_ref[0] = jnp.where(s == 0.0, 1.0, 0.0)


def _pool_call(masks_flat):
    return pl.pallas_call(
        _pool_body,
        grid=(BM,),
        in_specs=[pl.BlockSpec((1, HW, HW), lambda i: (i, 0, 0))],
        out_specs=pl.BlockSpec((1, G, G), lambda i: (i, 0, 0)),
        out_shape=jax.ShapeDtypeStruct((BM, G, G), F32),
    )(masks_flat)


def _stats_body(pooled_ref, emb_ref, cat_ref,
                mref_ref, flat_ref, cnt_ref, scat_ref):
    pooled = pooled_ref[...]
    keep = (pooled != 0.0).astype(F32)
    cat_row = cat_ref[...]

    sums = []
    for b in range(B):
        kb = keep[b * M:(b + 1) * M]
        sums.append(jnp.dot(kb, emb_ref[b], preferred_element_type=F32,
                            precision=lax.Precision.HIGHEST))
    sum_per_mask = jnp.concatenate(sums, axis=0)

    cnt_col = jnp.dot(keep, jnp.ones((P, 1), F32),
                      preferred_element_type=F32)

    cid = lax.broadcasted_iota(jnp.int32, (C, BM), 0)
    onehot = (jnp.broadcast_to(cat_row, (C, BM)) == cid).astype(F32)

    cat_sum = jnp.dot(onehot, sum_per_mask, preferred_element_type=F32,
                      precision=lax.Precision.HIGHEST)
    cat_cnt = jnp.dot(onehot, cnt_col, preferred_element_type=F32)
    mean = cat_sum / jnp.maximum(cat_cnt, 1.0)
    nrm = jnp.sqrt(jnp.sum(mean * mean, axis=1, keepdims=True))
    mref_ref[...] = mean / jnp.maximum(nrm, 1e-12)

    counts_col = jnp.dot(onehot, jnp.ones((BM, 1), F32),
                         preferred_element_type=F32)
    cnt_ref[...] = counts_col.astype(jnp.int32)

    lessc = jnp.dot((lax.broadcasted_iota(jnp.int32, (C, C), 1) <
                     lax.broadcasted_iota(jnp.int32, (C, C), 0)).astype(F32),
                    counts_col, preferred_element_type=F32)
    less_row = jnp.sum(onehot * lessc, axis=0, keepdims=True)
    u_strict = (lax.broadcasted_iota(jnp.int32, (BM, BM), 0) <
                lax.broadcasted_iota(jnp.int32, (BM, BM), 1)).astype(F32)
    prefix = jnp.dot(onehot, u_strict, preferred_element_type=F32)
    eqb_row = jnp.sum(onehot * prefix, axis=0, keepdims=True)
    rank_row = less_row + eqb_row

    rowj = lax.broadcasted_iota(jnp.int32, (BM, BM), 0)
    pm = (jnp.broadcast_to(rank_row, (BM, BM)) ==
          rowj.astype(F32)).astype(F32)
    flat_ref[...] = jnp.dot(pm, pooled, preferred_element_type=F32)

    rank_b = jnp.dot(
        (lax.broadcasted_iota(jnp.int32, (B, BM), 1) //
         M == lax.broadcasted_iota(jnp.int32, (B, BM), 0)).astype(F32) *
        jnp.broadcast_to(rank_row, (B, BM)),
        (lax.broadcasted_iota(jnp.int32, (BM, M), 0) % M ==
         lax.broadcasted_iota(jnp.int32, (BM, M), 1)).astype(F32),
        preferred_element_type=F32)
    rep_row = (lax.broadcasted_iota(jnp.int32, (BM, B), 0) // M ==
               lax.broadcasted_iota(jnp.int32, (BM, B), 1)).astype(F32)
    rep_col = (lax.broadcasted_iota(jnp.int32, (M, P), 1) // _ROWS ==
               lax.broadcasted_iota(jnp.int32, (M, P), 0)).astype(F32)
    rank_val = jnp.dot(jnp.dot(rep_row, rank_b, preferred_element_type=F32),
                       rep_col, preferred_element_type=F32)
    w_iota = lax.broadcasted_iota(jnp.int32, (BM, P), 0)
    q_iota = lax.broadcasted_iota(jnp.int32, (BM, P), 1)
    scat_ref[...] = (rank_val.astype(jnp.int32) * P +
                     (w_iota % M) * _ROWS + q_iota % _ROWS)


def _stats_call(pooled2, embeddings, cat_2d):
    return pl.pallas_call(
        _stats_body,
        out_shape=(
            jax.ShapeDtypeStruct((C, D), F32),
            jax.ShapeDtypeStruct((BM, P), F32),
            jax.ShapeDtypeStruct((C, 1), jnp.int32),
            jax.ShapeDtypeStruct((BM, P), jnp.int32),
        ),
    )(pooled2, embeddings, cat_2d)


_NC, _NS = 2, 16
_ROWS = P // M


def _gather_body(idx_hbm, emb_hbm, out_hbm, idx_v, buf, isem, osem):
    wid = lax.axis_index("s") * _NC + lax.axis_index("c")
    pltpu.sync_copy(idx_hbm.at[wid], idx_v)
    base = (wid // M) * P + (wid % M) * _ROWS
    pltpu.async_copy(emb_hbm.at[pl.ds(base, _ROWS)], buf, isem).wait()
    hs = []
    for j in range(M):
        hs.append(pltpu.async_copy(buf, out_hbm.at[idx_v.at[j]], osem))
    for h in hs:
        h.wait()


def _gather_call(idx3d, emb2d):
    mesh = plsc.VectorSubcoreMesh(core_axis_name="c", subcore_axis_name="s")
    fn = pl.kernel(
        _gather_body,
        mesh=mesh,
        out_type=jax.ShapeDtypeStruct((BM * P, D), F32),
        scratch_types=[
            pltpu.VMEM((M, _ROWS), jnp.int32),
            pltpu.VMEM((_ROWS, D), F32),
            pltpu.SemaphoreType.DMA,
            pltpu.SemaphoreType.DMA,
        ],
    )
    return fn(idx3d, emb2d)


def kernel(embeddings, masks, category_ids):
    cat_flat = category_ids.reshape(BM)
    masks_flat = masks.reshape(BM, HW, HW)

    pooled = _pool_call(masks_flat).reshape(BM, P)
    masked_ref, flat_sorted, counts_col, scat = _stats_call(
        pooled, embeddings, cat_flat.reshape(1, BM))
    ref_emb = _gather_call(scat.reshape(BM, M, _ROWS),
                           embeddings.reshape(B * P, D))

    return (masked_ref,
            flat_sorted.reshape(BM * P),
            ref_emb.reshape(BM * P, D),
            counts_col.reshape(C))


# speedup vs baseline: 1.2905x; 1.2871x over previous
import functools

import jax
import jax.numpy as jnp
from jax import lax
from jax.experimental import pallas as pl
from jax.experimental.pallas import tpu as pltpu
from jax.experimental.pallas import tpu_sc as plsc

B, M, HW, PATCH, D, C = 4, 8, 512, 16, 768, 8
G = HW // PATCH
P = G * G
BM = B * M
F32 = jnp.float32


_PK = 2


def _pool_body(mask_ref, pooled_ref):
    r = lax.broadcasted_iota(jnp.int32, (HW, G), 0) // PATCH
    c = lax.broadcasted_iota(jnp.int32, (HW, G), 1)
    a = (r == c).astype(F32)
    for u in range(_PK):
        inv = 1.0 - mask_ref[u]
        t = jnp.dot(inv, a, preferred_element_type=F32)
        s = lax.dot_general(a, t, (((0,), (0,)), ((), ())),
                            preferred_element_type=F32)
        pooled_ref[u] = jnp.where(s == 0.0, 1.0, 0.0)


def _pool_call(masks_flat):
    return pl.pallas_call(
        _pool_body,
        grid=(BM // _PK,),
        in_specs=[pl.BlockSpec((_PK, HW, HW), lambda i: (i, 0, 0))],
        out_specs=pl.BlockSpec((_PK, G, G), lambda i: (i, 0, 0)),
        out_shape=jax.ShapeDtypeStruct((BM, G, G), F32),
    )(masks_flat)


def _stats_body(pooled_ref, emb_ref, cat_ref, mref_ref, flat_ref, cnt_ref):
    pooled = pooled_ref[...]
    keep = (pooled != 0.0).astype(F32)
    cat_row = cat_ref[...]

    sums = []
    for b in range(B):
        kb = keep[b * M:(b + 1) * M]
        sums.append(jnp.dot(kb, emb_ref[b], preferred_element_type=F32,
                            precision=lax.Precision.HIGHEST))
    sum_per_mask = jnp.concatenate(sums, axis=0)

    cnt_col = jnp.dot(keep, jnp.ones((P, 1), F32),
                      preferred_element_type=F32)

    cid = lax.broadcasted_iota(jnp.int32, (C, BM), 0)
    onehot = (jnp.broadcast_to(cat_row, (C, BM)) == cid).astype(F32)

    cat_sum = jnp.dot(onehot, sum_per_mask, preferred_element_type=F32,
                      precision=lax.Precision.HIGHEST)
    cat_cnt = jnp.dot(onehot, cnt_col, preferred_element_type=F32)
    mean = cat_sum / jnp.maximum(cat_cnt, 1.0)
    nrm = jnp.sqrt(jnp.sum(mean * mean, axis=1, keepdims=True))
    mref_ref[...] = mean / jnp.maximum(nrm, 1e-12)

    counts_col = jnp.dot(onehot, jnp.ones((BM, 1), F32),
                         preferred_element_type=F32)
    cnt_ref[...] = counts_col.astype(jnp.int32)

    lessc = jnp.dot((lax.broadcasted_iota(jnp.int32, (C, C), 1) <
                     lax.broadcasted_iota(jnp.int32, (C, C), 0)).astype(F32),
                    counts_col, preferred_element_type=F32)
    less_row = jnp.sum(onehot * lessc, axis=0, keepdims=True)
    u_strict = (lax.broadcasted_iota(jnp.int32, (BM, BM), 0) <
                lax.broadcasted_iota(jnp.int32, (BM, BM), 1)).astype(F32)
    prefix = jnp.dot(onehot, u_strict, preferred_element_type=F32)
    eqb_row = jnp.sum(onehot * prefix, axis=0, keepdims=True)
    rank_row = less_row + eqb_row

    rowj = lax.broadcasted_iota(jnp.int32, (BM, BM), 0)
    pm = (jnp.broadcast_to(rank_row, (BM, BM)) ==
          rowj.astype(F32)).astype(F32)
    flat_ref[...] = jnp.dot(pm, pooled, preferred_element_type=F32)


def _stats_call(pooled2, embeddings, cat_2d):
    return pl.pallas_call(
        _stats_body,
        out_shape=(
            jax.ShapeDtypeStruct((C, D), F32),
            jax.ShapeDtypeStruct((BM, P), F32),
            jax.ShapeDtypeStruct((C, 1), jnp.int32),
        ),
    )(pooled2, embeddings, cat_2d)


_NC, _NS = 2, 16
_CHUNK = 64


def _order_body(cat_ref, idx_ref):
    cat_row = cat_ref[...]
    cid = lax.broadcasted_iota(jnp.int32, (C, BM), 0)
    onehot = (jnp.broadcast_to(cat_row, (C, BM)) == cid).astype(F32)
    counts_col = jnp.dot(onehot, jnp.ones((BM, 1), F32),
                         preferred_element_type=F32)
    lessc = jnp.dot((lax.broadcasted_iota(jnp.int32, (C, C), 1) <
                     lax.broadcasted_iota(jnp.int32, (C, C), 0)).astype(F32),
                    counts_col, preferred_element_type=F32)
    less_row = jnp.sum(onehot * lessc, axis=0, keepdims=True)
    u_strict = (lax.broadcasted_iota(jnp.int32, (BM, BM), 0) <
                lax.broadcasted_iota(jnp.int32, (BM, BM), 1)).astype(F32)
    prefix = jnp.dot(onehot, u_strict, preferred_element_type=F32)
    eqb_row = jnp.sum(onehot * prefix, axis=0, keepdims=True)
    rank_row = less_row + eqb_row
    rowj = lax.broadcasted_iota(jnp.int32, (BM, BM), 0)
    pm = (jnp.broadcast_to(rank_row, (BM, BM)) ==
          rowj.astype(F32)).astype(F32)
    rank_b = jnp.dot(
        (lax.broadcasted_iota(jnp.int32, (B, BM), 1) //
         M == lax.broadcasted_iota(jnp.int32, (B, BM), 0)).astype(F32) *
        jnp.broadcast_to(rank_row, (B, BM)),
        (lax.broadcasted_iota(jnp.int32, (BM, M), 0) % M ==
         lax.broadcasted_iota(jnp.int32, (BM, M), 1)).astype(F32),
        preferred_element_type=F32)
    rep_row = (lax.broadcasted_iota(jnp.int32, (BM, B), 0) // M ==
               lax.broadcasted_iota(jnp.int32, (BM, B), 1)).astype(F32)
    rep_col = (lax.broadcasted_iota(jnp.int32, (M, P), 1) // 128 ==
               lax.broadcasted_iota(jnp.int32, (M, P), 0)).astype(F32)
    rank_val = jnp.dot(jnp.dot(rep_row, rank_b, preferred_element_type=F32),
                       rep_col, preferred_element_type=F32)
    w_iota = lax.broadcasted_iota(jnp.int32, (BM, P), 0)
    q_iota = lax.broadcasted_iota(jnp.int32, (BM, P), 1)
    idx_ref[...] = (rank_val.astype(jnp.int32) * P +
                    (w_iota % M) * 128 + q_iota % 128)


def _order_call(cat_2d):
    return pl.pallas_call(
        _order_body,
        out_shape=jax.ShapeDtypeStruct((BM, P), jnp.int32),
    )(cat_2d)


_ROWS = P // M


def _gather_body(idx_hbm, emb_hbm, out_hbm, idx_v, buf, isem, osem):
    wid = lax.axis_index("s") * _NC + lax.axis_index("c")
    pltpu.sync_copy(idx_hbm.at[wid], idx_v)
    base = (wid // M) * P + (wid % M) * _ROWS
    pltpu.async_copy(emb_hbm.at[pl.ds(base, _ROWS)], buf, isem).wait()
    hs = []
    for j in range(M):
        hs.append(pltpu.async_copy(buf, out_hbm.at[idx_v.at[j]], osem))
    for h in hs:
        h.wait()


def _gather_call(idx3d, emb2d):
    mesh = plsc.VectorSubcoreMesh(core_axis_name="c", subcore_axis_name="s")
    fn = pl.kernel(
        _gather_body,
        mesh=mesh,
        out_type=jax.ShapeDtypeStruct((BM * P, D), F32),
        scratch_types=[
            pltpu.VMEM((M, _ROWS), jnp.int32),
            pltpu.VMEM((_ROWS, D), F32),
            pltpu.SemaphoreType.DMA,
            pltpu.SemaphoreType.DMA,
        ],
    )
    return fn(idx3d, emb2d)


def kernel(embeddings, masks, category_ids):
    cat_flat = category_ids.reshape(BM)
    masks_flat = masks.reshape(BM, HW, HW)

    idx3d = _order_call(cat_flat.reshape(1, BM)).reshape(BM, M, _ROWS)
    ref_emb = _gather_call(idx3d, embeddings.reshape(B * P, D))

    pooled = _pool_call(masks_flat).reshape(BM, P)
    masked_ref, flat_sorted, counts_col = _stats_call(
        pooled, embeddings, cat_flat.reshape(1, BM))

    return (masked_ref,
            flat_sorted.reshape(BM * P),
            ref_emb.reshape(BM * P, D),
            counts_col.reshape(C))


# speedup vs baseline: 1.3772x; 1.0672x over previous
import functools

import jax
import jax.numpy as jnp
from jax import lax
from jax.experimental import pallas as pl
from jax.experimental.pallas import tpu as pltpu
from jax.experimental.pallas import tpu_sc as plsc

B, M, HW, PATCH, D, C = 4, 8, 512, 16, 768, 8
G = HW // PATCH
P = G * G
BM = B * M
F32 = jnp.float32


_PK = 4


def _pool_body(mask_ref, pooled_ref):
    r = lax.broadcasted_iota(jnp.int32, (HW, G), 0) // PATCH
    c = lax.broadcasted_iota(jnp.int32, (HW, G), 1)
    a = (r == c).astype(F32)
    for u in range(_PK):
        inv = 1.0 - mask_ref[u]
        t = jnp.dot(inv, a, preferred_element_type=F32)
        s = lax.dot_general(a, t, (((0,), (0,)), ((), ())),
                            preferred_element_type=F32)
        pooled_ref[u] = jnp.where(s == 0.0, 1.0, 0.0)


def _pool_call(masks_flat):
    return pl.pallas_call(
        _pool_body,
        grid=(BM // _PK,),
        in_specs=[pl.BlockSpec((_PK, HW, HW), lambda i: (i, 0, 0))],
        out_specs=pl.BlockSpec((_PK, G, G), lambda i: (i, 0, 0)),
        out_shape=jax.ShapeDtypeStruct((BM, G, G), F32),
    )(masks_flat)


def _stats_body(pooled_ref, emb_ref, cat_ref, mref_ref, flat_ref, cnt_ref):
    pooled = pooled_ref[...]
    keep = (pooled != 0.0).astype(F32)
    cat_row = cat_ref[...]

    sums = []
    for b in range(B):
        kb = keep[b * M:(b + 1) * M]
        sums.append(jnp.dot(kb, emb_ref[b], preferred_element_type=F32,
                            precision=lax.Precision.HIGHEST))
    sum_per_mask = jnp.concatenate(sums, axis=0)

    cnt_col = jnp.dot(keep, jnp.ones((P, 1), F32),
                      preferred_element_type=F32)

    cid = lax.broadcasted_iota(jnp.int32, (C, BM), 0)
    onehot = (jnp.broadcast_to(cat_row, (C, BM)) == cid).astype(F32)

    cat_sum = jnp.dot(onehot, sum_per_mask, preferred_element_type=F32,
                      precision=lax.Precision.HIGHEST)
    cat_cnt = jnp.dot(onehot, cnt_col, preferred_element_type=F32)
    mean = cat_sum / jnp.maximum(cat_cnt, 1.0)
    nrm = jnp.sqrt(jnp.sum(mean * mean, axis=1, keepdims=True))
    mref_ref[...] = mean / jnp.maximum(nrm, 1e-12)

    counts_col = jnp.dot(onehot, jnp.ones((BM, 1), F32),
                         preferred_element_type=F32)
    cnt_ref[...] = counts_col.astype(jnp.int32)

    lessc = jnp.dot((lax.broadcasted_iota(jnp.int32, (C, C), 1) <
                     lax.broadcasted_iota(jnp.int32, (C, C), 0)).astype(F32),
                    counts_col, preferred_element_type=F32)
    less_row = jnp.sum(onehot * lessc, axis=0, keepdims=True)
    u_strict = (lax.broadcasted_iota(jnp.int32, (BM, BM), 0) <
                lax.broadcasted_iota(jnp.int32, (BM, BM), 1)).astype(F32)
    prefix = jnp.dot(onehot, u_strict, preferred_element_type=F32)
    eqb_row = jnp.sum(onehot * prefix, axis=0, keepdims=True)
    rank_row = less_row + eqb_row

    rowj = lax.broadcasted_iota(jnp.int32, (BM, BM), 0)
    pm = (jnp.broadcast_to(rank_row, (BM, BM)) ==
          rowj.astype(F32)).astype(F32)
    flat_ref[...] = jnp.dot(pm, pooled, preferred_element_type=F32)


def _stats_call(pooled2, embeddings, cat_2d):
    return pl.pallas_call(
        _stats_body,
        out_shape=(
            jax.ShapeDtypeStruct((C, D), F32),
            jax.ShapeDtypeStruct((BM, P), F32),
            jax.ShapeDtypeStruct((C, 1), jnp.int32),
        ),
    )(pooled2, embeddings, cat_2d)


_NC, _NS = 2, 16
_CHUNK = 64


def _order_body(cat_ref, idx_ref):
    cat_row = cat_ref[...]
    cid = lax.broadcasted_iota(jnp.int32, (C, BM), 0)
    onehot = (jnp.broadcast_to(cat_row, (C, BM)) == cid).astype(F32)
    counts_col = jnp.dot(onehot, jnp.ones((BM, 1), F32),
                         preferred_element_type=F32)
    lessc = jnp.dot((lax.broadcasted_iota(jnp.int32, (C, C), 1) <
                     lax.broadcasted_iota(jnp.int32, (C, C), 0)).astype(F32),
                    counts_col, preferred_element_type=F32)
    less_row = jnp.sum(onehot * lessc, axis=0, keepdims=True)
    u_strict = (lax.broadcasted_iota(jnp.int32, (BM, BM), 0) <
                lax.broadcasted_iota(jnp.int32, (BM, BM), 1)).astype(F32)
    prefix = jnp.dot(onehot, u_strict, preferred_element_type=F32)
    eqb_row = jnp.sum(onehot * prefix, axis=0, keepdims=True)
    rank_row = less_row + eqb_row
    rowj = lax.broadcasted_iota(jnp.int32, (BM, BM), 0)
    pm = (jnp.broadcast_to(rank_row, (BM, BM)) ==
          rowj.astype(F32)).astype(F32)
    rank_b = jnp.dot(
        (lax.broadcasted_iota(jnp.int32, (B, BM), 1) //
         M == lax.broadcasted_iota(jnp.int32, (B, BM), 0)).astype(F32) *
        jnp.broadcast_to(rank_row, (B, BM)),
        (lax.broadcasted_iota(jnp.int32, (BM, M), 0) % M ==
         lax.broadcasted_iota(jnp.int32, (BM, M), 1)).astype(F32),
        preferred_element_type=F32)
    rep_row = (lax.broadcasted_iota(jnp.int32, (BM, B), 0) // M ==
               lax.broadcasted_iota(jnp.int32, (BM, B), 1)).astype(F32)
    rep_col = (lax.broadcasted_iota(jnp.int32, (M, P), 1) // 128 ==
               lax.broadcasted_iota(jnp.int32, (M, P), 0)).astype(F32)
    rank_val = jnp.dot(jnp.dot(rep_row, rank_b, preferred_element_type=F32),
                       rep_col, preferred_element_type=F32)
    w_iota = lax.broadcasted_iota(jnp.int32, (BM, P), 0)
    q_iota = lax.broadcasted_iota(jnp.int32, (BM, P), 1)
    idx_ref[...] = (rank_val.astype(jnp.int32) * P +
                    (w_iota % M) * 128 + q_iota % 128)


def _order_call(cat_2d):
    return pl.pallas_call(
        _order_body,
        out_shape=jax.ShapeDtypeStruct((BM, P), jnp.int32),
    )(cat_2d)


_ROWS = P // M


def _gather_body(idx_hbm, emb_hbm, out_hbm, idx_v, buf, isem, osem):
    wid = lax.axis_index("s") * _NC + lax.axis_index("c")
    pltpu.sync_copy(idx_hbm.at[wid], idx_v)
    base = (wid // M) * P + (wid % M) * _ROWS
    pltpu.async_copy(emb_hbm.at[pl.ds(base, _ROWS)], buf, isem).wait()
    hs = []
    for j in range(M):
        hs.append(pltpu.async_copy(buf, out_hbm.at[idx_v.at[j]], osem))
    for h in hs:
        h.wait()


def _gather_call(idx3d, emb2d):
    mesh = plsc.VectorSubcoreMesh(core_axis_name="c", subcore_axis_name="s")
    fn = pl.kernel(
        _gather_body,
        mesh=mesh,
        out_type=jax.ShapeDtypeStruct((BM * P, D), F32),
        scratch_types=[
            pltpu.VMEM((M, _ROWS), jnp.int32),
            pltpu.VMEM((_ROWS, D), F32),
            pltpu.SemaphoreType.DMA,
            pltpu.SemaphoreType.DMA,
        ],
    )
    return fn(idx3d, emb2d)


def kernel(embeddings, masks, category_ids):
    cat_flat = category_ids.reshape(BM)
    masks_flat = masks.reshape(BM, HW, HW)

    idx3d = _order_call(cat_flat.reshape(1, BM)).reshape(BM, M, _ROWS)
    ref_emb = _gather_call(idx3d, embeddings.reshape(B * P, D))

    pooled = _pool_call(masks_flat).reshape(BM, P)
    masked_ref, flat_sorted, counts_col = _stats_call(
        pooled, embeddings, cat_flat.reshape(1, BM))

    return (masked_ref,
            flat_sorted.reshape(BM * P),
            ref_emb.reshape(BM * P, D),
            counts_col.reshape(C))


# speedup vs baseline: 1.4157x; 1.0279x over previous
import functools

import jax
import jax.numpy as jnp
from jax import lax
from jax.experimental import pallas as pl
from jax.experimental.pallas import tpu as pltpu
from jax.experimental.pallas import tpu_sc as plsc

B, M, HW, PATCH, D, C = 4, 8, 512, 16, 768, 8
G = HW // PATCH
P = G * G
BM = B * M
F32 = jnp.float32


_PK = 8


def _pool_body(mask_ref, pooled_ref):
    r = lax.broadcasted_iota(jnp.int32, (HW, G), 0) // PATCH
    c = lax.broadcasted_iota(jnp.int32, (HW, G), 1)
    a = (r == c).astype(F32)
    for u in range(_PK):
        inv = 1.0 - mask_ref[u]
        t = jnp.dot(inv, a, preferred_element_type=F32)
        s = lax.dot_general(a, t, (((0,), (0,)), ((), ())),
                            preferred_element_type=F32)
        pooled_ref[u] = jnp.where(s == 0.0, 1.0, 0.0)


def _pool_call(masks_flat):
    return pl.pallas_call(
        _pool_body,
        grid=(BM // _PK,),
        in_specs=[pl.BlockSpec((_PK, HW, HW), lambda i: (i, 0, 0))],
        out_specs=pl.BlockSpec((_PK, G, G), lambda i: (i, 0, 0)),
        out_shape=jax.ShapeDtypeStruct((BM, G, G), F32),
    )(masks_flat)


def _stats_body(pooled_ref, emb_ref, cat_ref, mref_ref, flat_ref, cnt_ref):
    pooled = pooled_ref[...]
    keep = (pooled != 0.0).astype(F32)
    cat_row = cat_ref[...]

    sums = []
    for b in range(B):
        kb = keep[b * M:(b + 1) * M]
        sums.append(jnp.dot(kb, emb_ref[b], preferred_element_type=F32,
                            precision=lax.Precision.HIGHEST))
    sum_per_mask = jnp.concatenate(sums, axis=0)

    cnt_col = jnp.dot(keep, jnp.ones((P, 1), F32),
                      preferred_element_type=F32)

    cid = lax.broadcasted_iota(jnp.int32, (C, BM), 0)
    onehot = (jnp.broadcast_to(cat_row, (C, BM)) == cid).astype(F32)

    cat_sum = jnp.dot(onehot, sum_per_mask, preferred_element_type=F32,
                      precision=lax.Precision.HIGHEST)
    cat_cnt = jnp.dot(onehot, cnt_col, preferred_element_type=F32)
    mean = cat_sum / jnp.maximum(cat_cnt, 1.0)
    nrm = jnp.sqrt(jnp.sum(mean * mean, axis=1, keepdims=True))
    mref_ref[...] = mean / jnp.maximum(nrm, 1e-12)

    counts_col = jnp.dot(onehot, jnp.ones((BM, 1), F32),
                         preferred_element_type=F32)
    cnt_ref[...] = counts_col.astype(jnp.int32)

    lessc = jnp.dot((lax.broadcasted_iota(jnp.int32, (C, C), 1) <
                     lax.broadcasted_iota(jnp.int32, (C, C), 0)).astype(F32),
                    counts_col, preferred_element_type=F32)
    less_row = jnp.sum(onehot * lessc, axis=0, keepdims=True)
    u_strict = (lax.broadcasted_iota(jnp.int32, (BM, BM), 0) <
                lax.broadcasted_iota(jnp.int32, (BM, BM), 1)).astype(F32)
    prefix = jnp.dot(onehot, u_strict, preferred_element_type=F32)
    eqb_row = jnp.sum(onehot * prefix, axis=0, keepdims=True)
    rank_row = less_row + eqb_row

    rowj = lax.broadcasted_iota(jnp.int32, (BM, BM), 0)
    pm = (jnp.broadcast_to(rank_row, (BM, BM)) ==
          rowj.astype(F32)).astype(F32)
    flat_ref[...] = jnp.dot(pm, pooled, preferred_element_type=F32)


def _stats_call(pooled2, embeddings, cat_2d):
    return pl.pallas_call(
        _stats_body,
        out_shape=(
            jax.ShapeDtypeStruct((C, D), F32),
            jax.ShapeDtypeStruct((BM, P), F32),
            jax.ShapeDtypeStruct((C, 1), jnp.int32),
        ),
    )(pooled2, embeddings, cat_2d)


_NC, _NS = 2, 16
_CHUNK = 64


def _order_body(cat_ref, idx_ref):
    cat_row = cat_ref[...]
    cid = lax.broadcasted_iota(jnp.int32, (C, BM), 0)
    onehot = (jnp.broadcast_to(cat_row, (C, BM)) == cid).astype(F32)
    counts_col = jnp.dot(onehot, jnp.ones((BM, 1), F32),
                         preferred_element_type=F32)
    lessc = jnp.dot((lax.broadcasted_iota(jnp.int32, (C, C), 1) <
                     lax.broadcasted_iota(jnp.int32, (C, C), 0)).astype(F32),
                    counts_col, preferred_element_type=F32)
    less_row = jnp.sum(onehot * lessc, axis=0, keepdims=True)
    u_strict = (lax.broadcasted_iota(jnp.int32, (BM, BM), 0) <
                lax.broadcasted_iota(jnp.int32, (BM, BM), 1)).astype(F32)
    prefix = jnp.dot(onehot, u_strict, preferred_element_type=F32)
    eqb_row = jnp.sum(onehot * prefix, axis=0, keepdims=True)
    rank_row = less_row + eqb_row
    rowj = lax.broadcasted_iota(jnp.int32, (BM, BM), 0)
    pm = (jnp.broadcast_to(rank_row, (BM, BM)) ==
          rowj.astype(F32)).astype(F32)
    rank_b = jnp.dot(
        (lax.broadcasted_iota(jnp.int32, (B, BM), 1) //
         M == lax.broadcasted_iota(jnp.int32, (B, BM), 0)).astype(F32) *
        jnp.broadcast_to(rank_row, (B, BM)),
        (lax.broadcasted_iota(jnp.int32, (BM, M), 0) % M ==
         lax.broadcasted_iota(jnp.int32, (BM, M), 1)).astype(F32),
        preferred_element_type=F32)
    rep_row = (lax.broadcasted_iota(jnp.int32, (BM, B), 0) // M ==
               lax.broadcasted_iota(jnp.int32, (BM, B), 1)).astype(F32)
    rep_col = (lax.broadcasted_iota(jnp.int32, (M, P), 1) // 128 ==
               lax.broadcasted_iota(jnp.int32, (M, P), 0)).astype(F32)
    rank_val = jnp.dot(jnp.dot(rep_row, rank_b, preferred_element_type=F32),
                       rep_col, preferred_element_type=F32)
    w_iota = lax.broadcasted_iota(jnp.int32, (BM, P), 0)
    q_iota = lax.broadcasted_iota(jnp.int32, (BM, P), 1)
    idx_ref[...] = (rank_val.astype(jnp.int32) * P +
                    (w_iota % M) * 128 + q_iota % 128)


def _order_call(cat_2d):
    return pl.pallas_call(
        _order_body,
        out_shape=jax.ShapeDtypeStruct((BM, P), jnp.int32),
    )(cat_2d)


_ROWS = P // M


def _gather_body(idx_hbm, emb_hbm, out_hbm, idx_v, buf, isem, osem):
    wid = lax.axis_index("s") * _NC + lax.axis_index("c")
    pltpu.sync_copy(idx_hbm.at[wid], idx_v)
    base = (wid // M) * P + (wid % M) * _ROWS
    pltpu.async_copy(emb_hbm.at[pl.ds(base, _ROWS)], buf, isem).wait()
    hs = []
    for j in range(M):
        hs.append(pltpu.async_copy(buf, out_hbm.at[idx_v.at[j]], osem))
    for h in hs:
        h.wait()


def _gather_call(idx3d, emb2d):
    mesh = plsc.VectorSubcoreMesh(core_axis_name="c", subcore_axis_name="s")
    fn = pl.kernel(
        _gather_body,
        mesh=mesh,
        out_type=jax.ShapeDtypeStruct((BM * P, D), F32),
        scratch_types=[
            pltpu.VMEM((M, _ROWS), jnp.int32),
            pltpu.VMEM((_ROWS, D), F32),
            pltpu.SemaphoreType.DMA,
            pltpu.SemaphoreType.DMA,
        ],
    )
    return fn(idx3d, emb2d)


def kernel(embeddings, masks, category_ids):
    cat_flat = category_ids.reshape(BM)
    masks_flat = masks.reshape(BM, HW, HW)

    idx3d = _order_call(cat_flat.reshape(1, BM)).reshape(BM, M, _ROWS)
    ref_emb = _gather_call(idx3d, embeddings.reshape(B * P, D))

    pooled = _pool_call(masks_flat).reshape(BM, P)
    masked_ref, flat_sorted, counts_col = _stats_call(
        pooled, embeddings, cat_flat.reshape(1, BM))

    return (masked_ref,
            flat_sorted.reshape(BM * P),
            ref_emb.reshape(BM * P, D),
            counts_col.reshape(C))
